# Initial kernel scaffold; baseline (speedup 1.0000x reference)
import jax
import jax.numpy as jnp
from jax.experimental import pallas as pl


def kernel(token_ids, weight):
    raise NotImplementedError("write your pallas kernel here")


# speedup vs baseline: 4.9809x; 4.9809x over previous
import functools

import jax
import jax.numpy as jnp
from jax import lax
from jax.experimental import pallas as pl
from jax.experimental.pallas import tpu as pltpu
from jax.experimental.pallas import tpu_sc as plsc

NUM_CORES = 2
NUM_SUBCORES = 16
NUM_WORKERS = NUM_CORES * NUM_SUBCORES

CHUNK = 1600
NBUF = 2


@functools.lru_cache(maxsize=None)
def _make_kernel(B: int, V: int, D: int):
    assert B % (NUM_WORKERS * CHUNK) == 0
    b_per_w = B // NUM_WORKERS
    n_chunks = b_per_w // CHUNK
    assert n_chunks % NBUF == 0
    n_groups = n_chunks // NBUF

    mesh = plsc.VectorSubcoreMesh(
        core_axis_name="c", subcore_axis_name="s", num_cores=NUM_CORES,
        num_subcores=NUM_SUBCORES)

    @functools.partial(
        pl.kernel,
        out_type=jax.ShapeDtypeStruct((B, D), jnp.float32),
        mesh=mesh,
        scratch_types=[
            [pltpu.VMEM((CHUNK,), jnp.int32) for _ in range(NBUF)],
            [pltpu.VMEM((CHUNK, D), jnp.float32) for _ in range(NBUF)],
            [pltpu.SemaphoreType.DMA for _ in range(NBUF)],
            [pltpu.SemaphoreType.DMA for _ in range(NBUF)],
        ],
        compiler_params=pltpu.CompilerParams(use_tc_tiling_on_sc=False),
    )
    def gather_kernel(tok_hbm, table_hbm, out_hbm, idx_v, rows_v, gsem, wsem):
        wid = lax.axis_index("s") * NUM_CORES + lax.axis_index("c")
        base = wid * b_per_w

        def load_and_fire(i, b):
            pltpu.sync_copy(tok_hbm.at[pl.ds(base + i * CHUNK, CHUNK)],
                            idx_v[b])
            pltpu.async_copy(table_hbm.at[idx_v[b]], rows_v[b], gsem[b])

        def wait_gather(b):
            pltpu.make_async_copy(table_hbm.at[idx_v[b]], rows_v[b],
                                  gsem[b]).wait()

        def fire_write(i, b):
            pltpu.async_copy(rows_v[b],
                             out_hbm.at[pl.ds(base + i * CHUNK, CHUNK)],
                             wsem[b])

        def wait_write(i, b):
            pltpu.make_async_copy(rows_v[b],
                                  out_hbm.at[pl.ds(base + i * CHUNK, CHUNK)],
                                  wsem[b]).wait()

        for b in range(NBUF):
            load_and_fire(b, b)

        def group_body(g, carry):
            for b in range(NBUF):
                i = NBUF * g + b
                wait_gather(b)
                fire_write(i, b)
                wait_write(i, b)
                load_and_fire(i + NBUF, b)
            return carry

        lax.fori_loop(0, n_groups - 1, group_body, 0)

        for b in range(NBUF):
            i = NBUF * (n_groups - 1) + b
            wait_gather(b)
            fire_write(i, b)
        for b in range(NBUF):
            i = NBUF * (n_groups - 1) + b
            wait_write(i, b)

    return gather_kernel


def kernel(token_ids, weight):
    B = token_ids.shape[0] * token_ids.shape[1]
    V, D = weight.shape
    flat = jnp.reshape(token_ids, (B,)).astype(jnp.int32)
    out = _make_kernel(B, V, D)(flat, weight)
    return jnp.reshape(out, (*token_ids.shape, D))
